# baseline (speedup 1.0000x reference)
import functools

import jax
import jax.numpy as jnp
from jax import lax
from jax.experimental import pallas as pl
from jax.experimental.pallas import tpu as pltpu
from jax.experimental.pallas import tpu_sc as plsc

BATCH = 16384
NUM_FEATURES = 16384
NUM_CLASSES = 2

BM = 128


def _tc_linear_kernel(x_ref, w_ref, b_ref, o_ref):
    acc = jax.lax.dot_general(
        x_ref[...], w_ref[...], (((1,), (1,)), ((), ())),
        preferred_element_type=jnp.float32,
    )
    o_ref[...] = acc + b_ref[...]


def _tc_part(x, W, b2):
    rows = x.shape[0]
    return pl.pallas_call(
        _tc_linear_kernel,
        grid=(rows // BM,),
        in_specs=[
            pl.BlockSpec((BM, NUM_FEATURES), lambda i: (i, 0)),
            pl.BlockSpec((NUM_CLASSES, NUM_FEATURES), lambda i: (0, 0)),
            pl.BlockSpec((1, NUM_CLASSES), lambda i: (0, 0)),
        ],
        out_specs=pl.BlockSpec((BM, NUM_CLASSES), lambda i: (i, 0)),
        out_shape=jax.ShapeDtypeStruct((rows, NUM_CLASSES), jnp.float32),
    )(x, W, b2)


NC = 2
NS = 16
NW = NC * NS
L = 16

SC_ROWS = 2048
RPW = SC_ROWS // NW
RB = 4
GROUP = 16
KC = NUM_FEATURES // L


def _sc_linear(x_sc, W, binit):
    mesh = plsc.VectorSubcoreMesh(core_axis_name="c", subcore_axis_name="s")

    @functools.partial(
        pl.kernel,
        mesh=mesh,
        out_type=jax.ShapeDtypeStruct((SC_ROWS, NUM_CLASSES), jnp.float32),
        scratch_types=[
            pltpu.VMEM((NUM_CLASSES, NUM_FEATURES), jnp.float32),
            pltpu.VMEM((NUM_CLASSES, L), jnp.float32),
            pltpu.VMEM((RB, NUM_FEATURES), jnp.float32),
            pltpu.VMEM((NUM_CLASSES, GROUP, L), jnp.float32),
            pltpu.VMEM((RPW, NUM_CLASSES), jnp.float32),
        ],
        compiler_params=pltpu.CompilerParams(needs_layout_passes=False),
    )
    def k(x_hbm, w_hbm, binit_hbm, out_hbm, wbuf, bbuf, xbuf, pend, obuf):
        cid = lax.axis_index("c")
        sid = lax.axis_index("s")
        wid = sid * NC + cid
        base = wid * RPW

        pltpu.sync_copy(w_hbm, wbuf)
        pltpu.sync_copy(binit_hbm, bbuf)

        lane_ids = lax.iota(jnp.int32, L)

        def group_body(g, _):
            for sb in range(GROUP // RB):
                row_in_g = sb * RB
                row0 = base + g * GROUP + row_in_g
                pltpu.sync_copy(x_hbm.at[pl.ds(row0, RB)], xbuf)

                def chunk_body(kk, carry):
                    accs = list(carry)
                    off = kk * L
                    w0 = wbuf[0, pl.ds(off, L)]
                    w1 = wbuf[1, pl.ds(off, L)]
                    for r in range(RB):
                        xv = xbuf[r, pl.ds(off, L)]
                        accs[2 * r] = accs[2 * r] + xv * w0
                        accs[2 * r + 1] = accs[2 * r + 1] + xv * w1
                    return tuple(accs)

                init = []
                for r in range(RB):
                    init.append(bbuf[0, :])
                    init.append(bbuf[1, :])
                accs = lax.fori_loop(0, KC, chunk_body, tuple(init))

                for r in range(RB):
                    for c in range(NUM_CLASSES):
                        pend[c, row_in_g + r, :] = accs[2 * r + c]

            for c in range(NUM_CLASSES):
                cvec = jnp.full((L,), c, dtype=jnp.int32)
                total = jnp.zeros((L,), jnp.float32)
                for j in range(L):
                    col = plsc.load_gather(
                        pend, [cvec, lane_ids, jnp.full((L,), j, jnp.int32)]
                    )
                    total = total + col
                plsc.store_scatter(
                    obuf, [g * GROUP + lane_ids, cvec], total
                )
            return 0

        lax.fori_loop(0, RPW // GROUP, group_body, 0)
        pltpu.sync_copy(obuf, out_hbm.at[pl.ds(base, RPW)])

    return k(x_sc, W, binit)


def kernel(batch, W, b):
    b2 = b.reshape(1, NUM_CLASSES)
    binit = jnp.zeros((NUM_CLASSES, L), jnp.float32).at[:, 0].set(b)
    sc_out = _sc_linear(batch[:SC_ROWS], W, binit)
    tc_out = _tc_part(batch[SC_ROWS:], W, b2)
    return jnp.concatenate([sc_out, tc_out], axis=0)


# speedup vs baseline: 2.8849x; 2.8849x over previous
import functools

import jax
import jax.numpy as jnp
from jax import lax
from jax.experimental import pallas as pl
from jax.experimental.pallas import tpu as pltpu
from jax.experimental.pallas import tpu_sc as plsc

BATCH = 16384
NUM_FEATURES = 16384
NUM_CLASSES = 2

BM = 128


def _tc_linear_kernel(x_ref, w_ref, b_ref, o_ref):
    acc = jax.lax.dot_general(
        x_ref[...], w_ref[...], (((1,), (1,)), ((), ())),
        preferred_element_type=jnp.float32,
    )
    o_ref[...] = acc + b_ref[...]


def _tc_part(x, W, b2, row_off):
    off_blocks = row_off // BM
    return pl.pallas_call(
        _tc_linear_kernel,
        grid=((BATCH - row_off) // BM,),
        in_specs=[
            pl.BlockSpec((BM, NUM_FEATURES), lambda i: (i + off_blocks, 0)),
            pl.BlockSpec((NUM_CLASSES, NUM_FEATURES), lambda i: (0, 0)),
            pl.BlockSpec((1, NUM_CLASSES), lambda i: (0, 0)),
        ],
        out_specs=pl.BlockSpec((BM, NUM_CLASSES), lambda i: (i + off_blocks, 0)),
        out_shape=jax.ShapeDtypeStruct((BATCH, NUM_CLASSES), jnp.float32),
    )(x, W, b2)


NC = 2
NS = 16
NW = NC * NS
L = 16

SC_ROWS = 2048
RPW = SC_ROWS // NW
RB = 4
GROUP = 16
KC = NUM_FEATURES // L


def _sc_linear(x_sc, W, binit):
    mesh = plsc.VectorSubcoreMesh(core_axis_name="c", subcore_axis_name="s")

    @functools.partial(
        pl.kernel,
        mesh=mesh,
        out_type=jax.ShapeDtypeStruct((SC_ROWS, NUM_CLASSES), jnp.float32),
        scratch_types=[
            pltpu.VMEM((NUM_CLASSES, NUM_FEATURES), jnp.float32),
            pltpu.VMEM((NUM_CLASSES, L), jnp.float32),
            pltpu.VMEM((RB, NUM_FEATURES), jnp.float32),
            pltpu.VMEM((NUM_CLASSES, GROUP, L), jnp.float32),
            pltpu.VMEM((RPW, NUM_CLASSES), jnp.float32),
        ],
        compiler_params=pltpu.CompilerParams(needs_layout_passes=False),
    )
    def k(x_hbm, w_hbm, binit_hbm, out_hbm, wbuf, bbuf, xbuf, pend, obuf):
        cid = lax.axis_index("c")
        sid = lax.axis_index("s")
        wid = sid * NC + cid
        base = wid * RPW

        pltpu.sync_copy(w_hbm, wbuf)
        pltpu.sync_copy(binit_hbm, bbuf)

        lane_ids = lax.iota(jnp.int32, L)

        def group_body(g, _):
            for sb in range(GROUP // RB):
                row_in_g = sb * RB
                row0 = base + g * GROUP + row_in_g
                pltpu.sync_copy(x_hbm.at[pl.ds(row0, RB)], xbuf)

                def chunk_body(kk, carry):
                    accs = list(carry)
                    off = kk * L
                    w0 = wbuf[0, pl.ds(off, L)]
                    w1 = wbuf[1, pl.ds(off, L)]
                    for r in range(RB):
                        xv = xbuf[r, pl.ds(off, L)]
                        accs[2 * r] = accs[2 * r] + xv * w0
                        accs[2 * r + 1] = accs[2 * r + 1] + xv * w1
                    return tuple(accs)

                init = []
                for r in range(RB):
                    init.append(bbuf[0, :])
                    init.append(bbuf[1, :])
                accs = lax.fori_loop(0, KC, chunk_body, tuple(init))

                for r in range(RB):
                    for c in range(NUM_CLASSES):
                        pend[c, row_in_g + r, :] = accs[2 * r + c]

            for c in range(NUM_CLASSES):
                cvec = jnp.full((L,), c, dtype=jnp.int32)
                total = jnp.zeros((L,), jnp.float32)
                for j in range(L):
                    col = plsc.load_gather(
                        pend, [cvec, lane_ids, jnp.full((L,), j, jnp.int32)]
                    )
                    total = total + col
                plsc.store_scatter(
                    obuf, [g * GROUP + lane_ids, cvec], total
                )
            return 0

        lax.fori_loop(0, RPW // GROUP, group_body, 0)
        pltpu.sync_copy(obuf, out_hbm.at[pl.ds(base, RPW)])

    return k(x_sc, W, binit)


def kernel(batch, W, b):
    b2 = b.reshape(1, NUM_CLASSES)
    binit = jnp.zeros((NUM_CLASSES, L), jnp.float32).at[:, 0].set(b)
    sc_out = _sc_linear(batch, W, binit)
    tc_out = _tc_part(batch, W, b2, SC_ROWS)
    return lax.dynamic_update_slice(tc_out, sc_out, (0, 0))


# speedup vs baseline: 3.0762x; 1.0663x over previous
import jax
import jax.numpy as jnp
from jax.experimental import pallas as pl

BATCH = 16384
NUM_FEATURES = 16384
NUM_CLASSES = 2

BM = 256


def _linear_kernel(x_ref, w_ref, b_ref, o_ref):
    acc = jax.lax.dot_general(
        x_ref[...], w_ref[...], (((1,), (1,)), ((), ())),
        preferred_element_type=jnp.float32,
    )
    o_ref[...] = acc + b_ref[...]


def kernel(batch, W, b):
    b2 = b.reshape(1, NUM_CLASSES)
    return pl.pallas_call(
        _linear_kernel,
        grid=(BATCH // BM,),
        in_specs=[
            pl.BlockSpec((BM, NUM_FEATURES), lambda i: (i, 0)),
            pl.BlockSpec((NUM_CLASSES, NUM_FEATURES), lambda i: (0, 0)),
            pl.BlockSpec((1, NUM_CLASSES), lambda i: (0, 0)),
        ],
        out_specs=pl.BlockSpec((BM, NUM_CLASSES), lambda i: (i, 0)),
        out_shape=jax.ShapeDtypeStruct((BATCH, NUM_CLASSES), jnp.float32),
    )(batch, W, b2)
